# Initial kernel scaffold; baseline (speedup 1.0000x reference)
import jax
import jax.numpy as jnp
from jax.experimental import pallas as pl


def kernel(x, edge_index, edge_index_aux, W1, b1, Wk11, bk11, Wk12, bk12, Wk21, bk21, Wk22, bk22, Wk31, bk31, Wk32, bk32, Wr0, br0, Wr1, br1, Wr2, br2, Wr3, br3, wr0, wr1, wr2, wr3):
    raise NotImplementedError("write your pallas kernel here")


# speedup vs baseline: 12.2089x; 12.2089x over previous
import jax
import jax.numpy as jnp
from jax import lax
from jax.experimental import pallas as pl
from jax.experimental.pallas import tpu as pltpu
from jax.experimental.pallas import tpu_sc as plsc

_N = 10000
_E = 320000
_H = 128
_NC = 2
_NS = 16
_CHUNK = 125
_NCHUNK = (_E // _NS) // _CHUNK
_NPAD = 10240
_RPT = _NPAD // _NS
_ZR = 40
_IB = 40
_NBLK = _NCHUNK // _IB
_BR = 1000
_GRID = _N // _BR

_MESH = plsc.VectorSubcoreMesh(core_axis_name="c", subcore_axis_name="s")


_DW = 128


def _deg_body(dst_hbm, out_hbm, didx_v, ones_v, zbuf_v, acc_sh, sem):
    c = lax.axis_index("c")
    s = lax.axis_index("s")

    def _fill_ones(i, _):
        r = i // 8
        l = (i % 8) * 16
        ones_v[r, pl.ds(l, 16)] = jnp.ones((16,), jnp.float32)
        return 0
    lax.fori_loop(0, _CHUNK * 8, _fill_ones, 0)

    def _fill_zero(i, _):
        r = i // 8
        l = (i % 8) * 16
        zbuf_v[r, pl.ds(l, 16)] = jnp.zeros((16,), jnp.float32)
        return 0
    lax.fori_loop(0, _ZR * 8, _fill_zero, 0)

    def _zero(j, _):
        pltpu.sync_copy(zbuf_v, acc_sh.at[pl.ds(s * _RPT + j * _ZR, _ZR)])
        return 0
    lax.fori_loop(0, _RPT // _ZR, _zero, 0)
    plsc.subcore_barrier()

    def _blk(b, _):
        pltpu.sync_copy(dst_hbm.at[c, s, pl.ds(b * _IB, _IB)], didx_v)

        def _scat(j, _):
            pltpu.sync_copy(ones_v, acc_sh.at[didx_v.at[j]], add=True)
            return 0
        lax.fori_loop(0, _IB, _scat, 0)
        return 0
    lax.fori_loop(0, _NBLK, _blk, 0)
    plsc.subcore_barrier()

    pltpu.sync_copy(acc_sh.at[pl.ds(s * _RPT, _RPT)],
                    out_hbm.at[c, pl.ds(s * _RPT, _RPT)])


_deg_call = pl.kernel(
    _deg_body,
    out_type=jax.ShapeDtypeStruct((_NC, _NPAD, _DW), jnp.float32),
    mesh=_MESH,
    scratch_types=[
        pltpu.VMEM((_IB, _CHUNK), jnp.int32),
        pltpu.VMEM((_CHUNK, _DW), jnp.float32),
        pltpu.VMEM((_ZR, _DW), jnp.float32),
        pltpu.VMEM_SHARED((_NPAD, _DW), jnp.float32),
        pltpu.SemaphoreType.DMA,
    ],
)


def _prop_body(h_hbm, src_hbm, dst_hbm, out_hbm,
               sidx_v, didx_v, rows_v, zbuf_v, acc_sh, sem):
    c = lax.axis_index("c")
    s = lax.axis_index("s")

    def _fill_zero(i, _):
        r = i // 8
        l = (i % 8) * 16
        zbuf_v[r, pl.ds(l, 16)] = jnp.zeros((16,), jnp.float32)
        return 0
    lax.fori_loop(0, _ZR * 8, _fill_zero, 0)

    def _zero(j, _):
        pltpu.sync_copy(zbuf_v, acc_sh.at[pl.ds(s * _RPT + j * _ZR, _ZR)])
        return 0
    lax.fori_loop(0, _RPT // _ZR, _zero, 0)
    plsc.subcore_barrier()

    def _blk(b, _):
        pltpu.sync_copy(src_hbm.at[c, s, pl.ds(b * _IB, _IB)], sidx_v)
        pltpu.sync_copy(dst_hbm.at[c, s, pl.ds(b * _IB, _IB)], didx_v)

        def _edge(j, _):
            pltpu.async_copy(h_hbm.at[sidx_v.at[j]], rows_v, sem).wait()
            pltpu.sync_copy(rows_v, acc_sh.at[didx_v.at[j]], add=True)
            return 0
        lax.fori_loop(0, _IB, _edge, 0)
        return 0
    lax.fori_loop(0, _NBLK, _blk, 0)
    plsc.subcore_barrier()

    pltpu.sync_copy(acc_sh.at[pl.ds(s * _RPT, _RPT)],
                    out_hbm.at[c, pl.ds(s * _RPT, _RPT)])


_prop_call = pl.kernel(
    _prop_body,
    out_type=jax.ShapeDtypeStruct((_NC, _NPAD, _H), jnp.float32),
    mesh=_MESH,
    scratch_types=[
        pltpu.VMEM((_IB, _CHUNK), jnp.int32),
        pltpu.VMEM((_IB, _CHUNK), jnp.int32),
        pltpu.VMEM((_CHUNK, _H), jnp.float32),
        pltpu.VMEM((_ZR, _H), jnp.float32),
        pltpu.VMEM_SHARED((_NPAD, _H), jnp.float32),
        pltpu.SemaphoreType.DMA,
    ],
)


def _tc1_body(x_ref, w1_ref, b1_ref, wka_ref, wkb_ref, wr_ref, br_ref,
              wwr_ref, deg_ref, h_ref, dinv_ref, part_ref):
    r0 = jnp.maximum(
        jnp.dot(x_ref[...], w1_ref[...], preferred_element_type=jnp.float32)
        + b1_ref[...], 0.0)
    deg = deg_ref[...]
    dinv = jnp.where(deg > 0, 1.0 / jnp.sqrt(deg), 0.0)
    dinv_ref[...] = dinv
    h_ref[0] = dinv[:, 0:1] * jnp.dot(
        r0, wka_ref[...], preferred_element_type=jnp.float32)
    h_ref[1] = dinv[:, 1:2] * jnp.dot(
        r0, wkb_ref[...], preferred_element_type=jnp.float32)
    part_ref[...] = wwr_ref[0, 0] * (
        jnp.dot(r0, wr_ref[...], preferred_element_type=jnp.float32)
        + br_ref[0, 0])


_tc1_call = pl.pallas_call(
    _tc1_body,
    grid=(_GRID,),
    in_specs=[
        pl.BlockSpec((_BR, _H), lambda i: (i, 0)),
        pl.BlockSpec((_H, _H), lambda i: (0, 0)),
        pl.BlockSpec((1, _H), lambda i: (0, 0)),
        pl.BlockSpec((_H, _H), lambda i: (0, 0)),
        pl.BlockSpec((_H, _H), lambda i: (0, 0)),
        pl.BlockSpec((_H, 1), lambda i: (0, 0)),
        pl.BlockSpec((1, 1), lambda i: (0, 0)),
        pl.BlockSpec((1, 1), lambda i: (0, 0)),
        pl.BlockSpec((_BR, _NC), lambda i: (i, 0)),
    ],
    out_specs=[
        pl.BlockSpec((_NC, _BR, _H), lambda i: (0, i, 0)),
        pl.BlockSpec((_BR, _NC), lambda i: (i, 0)),
        pl.BlockSpec((_BR, 1), lambda i: (i, 0)),
    ],
    out_shape=[
        jax.ShapeDtypeStruct((_NC, _N, _H), jnp.float32),
        jax.ShapeDtypeStruct((_N, _NC), jnp.float32),
        jax.ShapeDtypeStruct((_N, 1), jnp.float32),
    ],
)


def _mid_body(s_ref, dinv_ref, ba_ref, bb_ref, wka_ref, wkb_ref, wr_ref,
              br_ref, wwr_ref, pin_ref, h_ref, part_ref):
    dinv = dinv_ref[...]
    left = dinv[:, 0:1] * s_ref[0] + ba_ref[...]
    right = dinv[:, 1:2] * s_ref[1] + bb_ref[...]
    rk = jnp.concatenate([left, right], axis=1)
    h_ref[0] = dinv[:, 0:1] * jnp.dot(
        rk, wka_ref[...], preferred_element_type=jnp.float32)
    h_ref[1] = dinv[:, 1:2] * jnp.dot(
        rk, wkb_ref[...], preferred_element_type=jnp.float32)
    part_ref[...] = pin_ref[...] + wwr_ref[0, 0] * (
        jnp.dot(rk, wr_ref[...], preferred_element_type=jnp.float32)
        + br_ref[0, 0])


_mid_call = pl.pallas_call(
    _mid_body,
    grid=(_GRID,),
    in_specs=[
        pl.BlockSpec((_NC, _BR, _H), lambda i: (0, i, 0)),
        pl.BlockSpec((_BR, _NC), lambda i: (i, 0)),
        pl.BlockSpec((1, _H), lambda i: (0, 0)),
        pl.BlockSpec((1, _H), lambda i: (0, 0)),
        pl.BlockSpec((2 * _H, _H), lambda i: (0, 0)),
        pl.BlockSpec((2 * _H, _H), lambda i: (0, 0)),
        pl.BlockSpec((2 * _H, 1), lambda i: (0, 0)),
        pl.BlockSpec((1, 1), lambda i: (0, 0)),
        pl.BlockSpec((1, 1), lambda i: (0, 0)),
        pl.BlockSpec((_BR, 1), lambda i: (i, 0)),
    ],
    out_specs=[
        pl.BlockSpec((_NC, _BR, _H), lambda i: (0, i, 0)),
        pl.BlockSpec((_BR, 1), lambda i: (i, 0)),
    ],
    out_shape=[
        jax.ShapeDtypeStruct((_NC, _N, _H), jnp.float32),
        jax.ShapeDtypeStruct((_N, 1), jnp.float32),
    ],
)


def _tc4_body(s_ref, dinv_ref, ba_ref, bb_ref, wr_ref, br_ref, wwr_ref,
              pin_ref, out_ref):
    dinv = dinv_ref[...]
    left = dinv[:, 0:1] * s_ref[0] + ba_ref[...]
    right = dinv[:, 1:2] * s_ref[1] + bb_ref[...]
    rk = jnp.concatenate([left, right], axis=1)
    out_ref[...] = pin_ref[...] + wwr_ref[0, 0] * (
        jnp.dot(rk, wr_ref[...], preferred_element_type=jnp.float32)
        + br_ref[0, 0])


_tc4_call = pl.pallas_call(
    _tc4_body,
    grid=(_GRID,),
    in_specs=[
        pl.BlockSpec((_NC, _BR, _H), lambda i: (0, i, 0)),
        pl.BlockSpec((_BR, _NC), lambda i: (i, 0)),
        pl.BlockSpec((1, _H), lambda i: (0, 0)),
        pl.BlockSpec((1, _H), lambda i: (0, 0)),
        pl.BlockSpec((2 * _H, 1), lambda i: (0, 0)),
        pl.BlockSpec((1, 1), lambda i: (0, 0)),
        pl.BlockSpec((1, 1), lambda i: (0, 0)),
        pl.BlockSpec((_BR, 1), lambda i: (i, 0)),
    ],
    out_specs=pl.BlockSpec((_BR, 1), lambda i: (i, 0)),
    out_shape=jax.ShapeDtypeStruct((_N, 1), jnp.float32),
)


def kernel(x, edge_index, edge_index_aux, W1, b1, Wk11, bk11, Wk12, bk12,
           Wk21, bk21, Wk22, bk22, Wk31, bk31, Wk32, bk32, Wr0, br0, Wr1,
           br1, Wr2, br2, Wr3, br3, wr0, wr1, wr2, wr3):
    s1, d1 = edge_index[0], edge_index[1]
    s2, d2 = edge_index_aux[0], edge_index_aux[1]
    dst4 = jnp.stack([d1, d2]).reshape(_NC, _NS, _NCHUNK, _CHUNK)
    src4 = jnp.stack([s1, s2 + _N]).reshape(_NC, _NS, _NCHUNK, _CHUNK)

    deg = _deg_call(dst4)[:, :_N, 0].T

    h, dinv, part = _tc1_call(
        x, W1, b1.reshape(1, _H), Wk11, Wk12, Wr0,
        br0.reshape(1, 1), wr0.reshape(1, 1), deg)
    S = _prop_call(h.reshape(_NC * _N, _H), src4, dst4)

    h, part = _mid_call(
        S, dinv, bk11.reshape(1, _H), bk12.reshape(1, _H), Wk21, Wk22, Wr1,
        br1.reshape(1, 1), wr1.reshape(1, 1), part)
    S = _prop_call(h.reshape(_NC * _N, _H), src4, dst4)

    h, part = _mid_call(
        S, dinv, bk21.reshape(1, _H), bk22.reshape(1, _H), Wk31, Wk32, Wr2,
        br2.reshape(1, 1), wr2.reshape(1, 1), part)
    S = _prop_call(h.reshape(_NC * _N, _H), src4, dst4)

    out = _tc4_call(
        S, dinv, bk31.reshape(1, _H), bk32.reshape(1, _H), Wr3,
        br3.reshape(1, 1), wr3.reshape(1, 1), part)
    return out


# speedup vs baseline: 17.7807x; 1.4564x over previous
import jax
import jax.numpy as jnp
from jax import lax
from jax.experimental import pallas as pl
from jax.experimental.pallas import tpu as pltpu
from jax.experimental.pallas import tpu_sc as plsc

_N = 10000
_E = 320000
_H = 128
_NC = 2
_NS = 16
_CHUNK = 125
_NCHUNK = (_E // _NS) // _CHUNK
_NPAD = 10240
_RPT = _NPAD // _NS
_ZR = 16
_IB = 40
_NBLK = _NCHUNK // _IB
_BR = 1000
_GRID = _N // _BR

_MESH = plsc.VectorSubcoreMesh(core_axis_name="c", subcore_axis_name="s")


_DW = 128


def _deg_body(dst_hbm, out_hbm, didx_v, ones_v, zbuf_v, acc_sh, sem):
    c = lax.axis_index("c")
    s = lax.axis_index("s")

    def _fill_ones(i, _):
        r = i // 8
        l = (i % 8) * 16
        ones_v[r, pl.ds(l, 16)] = jnp.ones((16,), jnp.float32)
        return 0
    lax.fori_loop(0, _CHUNK * 8, _fill_ones, 0)

    def _fill_zero(i, _):
        r = i // 8
        l = (i % 8) * 16
        zbuf_v[r, pl.ds(l, 16)] = jnp.zeros((16,), jnp.float32)
        return 0
    lax.fori_loop(0, _ZR * 8, _fill_zero, 0)

    def _zero(j, _):
        pltpu.sync_copy(zbuf_v, acc_sh.at[pl.ds(s * _RPT + j * _ZR, _ZR)])
        return 0
    lax.fori_loop(0, _RPT // _ZR, _zero, 0)
    plsc.subcore_barrier()

    def _blk(b, _):
        pltpu.sync_copy(dst_hbm.at[c, s, pl.ds(b * _IB, _IB)], didx_v)

        def _scat(j, _):
            pltpu.sync_copy(ones_v, acc_sh.at[didx_v.at[j]], add=True)
            return 0
        lax.fori_loop(0, _IB, _scat, 0)
        return 0
    lax.fori_loop(0, _NBLK, _blk, 0)
    plsc.subcore_barrier()

    pltpu.sync_copy(acc_sh.at[pl.ds(s * _RPT, _RPT)],
                    out_hbm.at[c, pl.ds(s * _RPT, _RPT)])


_deg_call = pl.kernel(
    _deg_body,
    out_type=jax.ShapeDtypeStruct((_NC, _NPAD, _DW), jnp.float32),
    mesh=_MESH,
    scratch_types=[
        pltpu.VMEM((_IB, _CHUNK), jnp.int32),
        pltpu.VMEM((_CHUNK, _DW), jnp.float32),
        pltpu.VMEM((_ZR, _DW), jnp.float32),
        pltpu.VMEM_SHARED((_NPAD, _DW), jnp.float32),
        pltpu.SemaphoreType.DMA,
    ],
)


def _prop_body(h_hbm, src_hbm, dst_hbm, out_hbm,
               sidx_v, didx_v, rows_v, zbuf_v, acc_sh, sem):
    c = lax.axis_index("c")
    s = lax.axis_index("s")

    def _fill_zero(i, _):
        r = i // 8
        l = (i % 8) * 16
        zbuf_v[r, pl.ds(l, 16)] = jnp.zeros((16,), jnp.float32)
        return 0
    lax.fori_loop(0, _ZR * 8, _fill_zero, 0)

    def _zero(j, _):
        pltpu.sync_copy(zbuf_v, acc_sh.at[pl.ds(s * _RPT + j * _ZR, _ZR)])
        return 0
    lax.fori_loop(0, _RPT // _ZR, _zero, 0)
    plsc.subcore_barrier()

    def _blk(b, _):
        pltpu.sync_copy(src_hbm.at[c, s, pl.ds(b * _IB, _IB)], sidx_v)
        pltpu.sync_copy(dst_hbm.at[c, s, pl.ds(b * _IB, _IB)], didx_v)
        pltpu.async_copy(h_hbm.at[sidx_v.at[0]], rows_v.at[0], sem)

        def _edge(j, _):
            @pl.when(j + 1 < _IB)
            def _pref():
                pltpu.async_copy(h_hbm.at[sidx_v.at[j + 1]],
                                 rows_v.at[(j + 1) % 2], sem)
            pltpu.make_async_copy(h_hbm.at[sidx_v.at[j]],
                                  rows_v.at[j % 2], sem).wait()
            pltpu.sync_copy(rows_v.at[j % 2], acc_sh.at[didx_v.at[j]], add=True)
            return 0
        lax.fori_loop(0, _IB, _edge, 0)
        return 0
    lax.fori_loop(0, _NBLK, _blk, 0)
    plsc.subcore_barrier()

    pltpu.sync_copy(acc_sh.at[pl.ds(s * _RPT, _RPT)],
                    out_hbm.at[c, pl.ds(s * _RPT, _RPT)])


_prop_call = pl.kernel(
    _prop_body,
    out_type=jax.ShapeDtypeStruct((_NC, _NPAD, _H), jnp.float32),
    mesh=_MESH,
    scratch_types=[
        pltpu.VMEM((_IB, _CHUNK), jnp.int32),
        pltpu.VMEM((_IB, _CHUNK), jnp.int32),
        pltpu.VMEM((2, _CHUNK, _H), jnp.float32),
        pltpu.VMEM((_ZR, _H), jnp.float32),
        pltpu.VMEM_SHARED((_NPAD, _H), jnp.float32),
        pltpu.SemaphoreType.DMA,
    ],
)


def _tc1_body(x_ref, w1_ref, b1_ref, wka_ref, wkb_ref, wr_ref, br_ref,
              wwr_ref, deg_ref, h_ref, dinv_ref, part_ref):
    r0 = jnp.maximum(
        jnp.dot(x_ref[...], w1_ref[...], preferred_element_type=jnp.float32)
        + b1_ref[...], 0.0)
    deg = deg_ref[...]
    dinv = jnp.where(deg > 0, 1.0 / jnp.sqrt(deg), 0.0)
    dinv_ref[...] = dinv
    h_ref[0] = dinv[:, 0:1] * jnp.dot(
        r0, wka_ref[...], preferred_element_type=jnp.float32)
    h_ref[1] = dinv[:, 1:2] * jnp.dot(
        r0, wkb_ref[...], preferred_element_type=jnp.float32)
    part_ref[...] = wwr_ref[0, 0] * (
        jnp.dot(r0, wr_ref[...], preferred_element_type=jnp.float32)
        + br_ref[0, 0])


_tc1_call = pl.pallas_call(
    _tc1_body,
    grid=(_GRID,),
    in_specs=[
        pl.BlockSpec((_BR, _H), lambda i: (i, 0)),
        pl.BlockSpec((_H, _H), lambda i: (0, 0)),
        pl.BlockSpec((1, _H), lambda i: (0, 0)),
        pl.BlockSpec((_H, _H), lambda i: (0, 0)),
        pl.BlockSpec((_H, _H), lambda i: (0, 0)),
        pl.BlockSpec((_H, 1), lambda i: (0, 0)),
        pl.BlockSpec((1, 1), lambda i: (0, 0)),
        pl.BlockSpec((1, 1), lambda i: (0, 0)),
        pl.BlockSpec((_BR, _NC), lambda i: (i, 0)),
    ],
    out_specs=[
        pl.BlockSpec((_NC, _BR, _H), lambda i: (0, i, 0)),
        pl.BlockSpec((_BR, _NC), lambda i: (i, 0)),
        pl.BlockSpec((_BR, 1), lambda i: (i, 0)),
    ],
    out_shape=[
        jax.ShapeDtypeStruct((_NC, _N, _H), jnp.float32),
        jax.ShapeDtypeStruct((_N, _NC), jnp.float32),
        jax.ShapeDtypeStruct((_N, 1), jnp.float32),
    ],
)


def _mid_body(s_ref, dinv_ref, ba_ref, bb_ref, wka_ref, wkb_ref, wr_ref,
              br_ref, wwr_ref, pin_ref, h_ref, part_ref):
    dinv = dinv_ref[...]
    left = dinv[:, 0:1] * s_ref[0] + ba_ref[...]
    right = dinv[:, 1:2] * s_ref[1] + bb_ref[...]
    rk = jnp.concatenate([left, right], axis=1)
    h_ref[0] = dinv[:, 0:1] * jnp.dot(
        rk, wka_ref[...], preferred_element_type=jnp.float32)
    h_ref[1] = dinv[:, 1:2] * jnp.dot(
        rk, wkb_ref[...], preferred_element_type=jnp.float32)
    part_ref[...] = pin_ref[...] + wwr_ref[0, 0] * (
        jnp.dot(rk, wr_ref[...], preferred_element_type=jnp.float32)
        + br_ref[0, 0])


_mid_call = pl.pallas_call(
    _mid_body,
    grid=(_GRID,),
    in_specs=[
        pl.BlockSpec((_NC, _BR, _H), lambda i: (0, i, 0)),
        pl.BlockSpec((_BR, _NC), lambda i: (i, 0)),
        pl.BlockSpec((1, _H), lambda i: (0, 0)),
        pl.BlockSpec((1, _H), lambda i: (0, 0)),
        pl.BlockSpec((2 * _H, _H), lambda i: (0, 0)),
        pl.BlockSpec((2 * _H, _H), lambda i: (0, 0)),
        pl.BlockSpec((2 * _H, 1), lambda i: (0, 0)),
        pl.BlockSpec((1, 1), lambda i: (0, 0)),
        pl.BlockSpec((1, 1), lambda i: (0, 0)),
        pl.BlockSpec((_BR, 1), lambda i: (i, 0)),
    ],
    out_specs=[
        pl.BlockSpec((_NC, _BR, _H), lambda i: (0, i, 0)),
        pl.BlockSpec((_BR, 1), lambda i: (i, 0)),
    ],
    out_shape=[
        jax.ShapeDtypeStruct((_NC, _N, _H), jnp.float32),
        jax.ShapeDtypeStruct((_N, 1), jnp.float32),
    ],
)


def _tc4_body(s_ref, dinv_ref, ba_ref, bb_ref, wr_ref, br_ref, wwr_ref,
              pin_ref, out_ref):
    dinv = dinv_ref[...]
    left = dinv[:, 0:1] * s_ref[0] + ba_ref[...]
    right = dinv[:, 1:2] * s_ref[1] + bb_ref[...]
    rk = jnp.concatenate([left, right], axis=1)
    out_ref[...] = pin_ref[...] + wwr_ref[0, 0] * (
        jnp.dot(rk, wr_ref[...], preferred_element_type=jnp.float32)
        + br_ref[0, 0])


_tc4_call = pl.pallas_call(
    _tc4_body,
    grid=(_GRID,),
    in_specs=[
        pl.BlockSpec((_NC, _BR, _H), lambda i: (0, i, 0)),
        pl.BlockSpec((_BR, _NC), lambda i: (i, 0)),
        pl.BlockSpec((1, _H), lambda i: (0, 0)),
        pl.BlockSpec((1, _H), lambda i: (0, 0)),
        pl.BlockSpec((2 * _H, 1), lambda i: (0, 0)),
        pl.BlockSpec((1, 1), lambda i: (0, 0)),
        pl.BlockSpec((1, 1), lambda i: (0, 0)),
        pl.BlockSpec((_BR, 1), lambda i: (i, 0)),
    ],
    out_specs=pl.BlockSpec((_BR, 1), lambda i: (i, 0)),
    out_shape=jax.ShapeDtypeStruct((_N, 1), jnp.float32),
)


def kernel(x, edge_index, edge_index_aux, W1, b1, Wk11, bk11, Wk12, bk12,
           Wk21, bk21, Wk22, bk22, Wk31, bk31, Wk32, bk32, Wr0, br0, Wr1,
           br1, Wr2, br2, Wr3, br3, wr0, wr1, wr2, wr3):
    s1, d1 = edge_index[0], edge_index[1]
    s2, d2 = edge_index_aux[0], edge_index_aux[1]
    dst4 = jnp.stack([d1, d2]).reshape(_NC, _NS, _NCHUNK, _CHUNK)
    src4 = jnp.stack([s1, s2 + _N]).reshape(_NC, _NS, _NCHUNK, _CHUNK)

    deg = _deg_call(dst4)[:, :_N, 0].T

    h, dinv, part = _tc1_call(
        x, W1, b1.reshape(1, _H), Wk11, Wk12, Wr0,
        br0.reshape(1, 1), wr0.reshape(1, 1), deg)
    S = _prop_call(h.reshape(_NC * _N, _H), src4, dst4)

    h, part = _mid_call(
        S, dinv, bk11.reshape(1, _H), bk12.reshape(1, _H), Wk21, Wk22, Wr1,
        br1.reshape(1, 1), wr1.reshape(1, 1), part)
    S = _prop_call(h.reshape(_NC * _N, _H), src4, dst4)

    h, part = _mid_call(
        S, dinv, bk21.reshape(1, _H), bk22.reshape(1, _H), Wk31, Wk32, Wr2,
        br2.reshape(1, 1), wr2.reshape(1, 1), part)
    S = _prop_call(h.reshape(_NC * _N, _H), src4, dst4)

    out = _tc4_call(
        S, dinv, bk31.reshape(1, _H), bk32.reshape(1, _H), Wr3,
        br3.reshape(1, 1), wr3.reshape(1, 1), part)
    return out


# speedup vs baseline: 17.8663x; 1.0048x over previous
import jax
import jax.numpy as jnp
from jax import lax
from jax.experimental import pallas as pl
from jax.experimental.pallas import tpu as pltpu
from jax.experimental.pallas import tpu_sc as plsc

_N = 10000
_E = 320000
_H = 128
_NC = 2
_NS = 16
_CHUNK = 125
_NCHUNK = (_E // _NS) // _CHUNK
_NPAD = 10240
_RPT = _NPAD // _NS
_ZR = 16
_IB = 40
_NBLK = _NCHUNK // _IB
_BR = 1000
_GRID = _N // _BR

_MESH = plsc.VectorSubcoreMesh(core_axis_name="c", subcore_axis_name="s")


_DW = 128


def _deg_body(dst_hbm, out_hbm, didx_v, ones_v, zbuf_v, acc_sh, sem):
    c = lax.axis_index("c")
    s = lax.axis_index("s")

    _LPR = _DW // 16

    def _fill_ones(i, _):
        ones_v[i // _LPR, pl.ds((i % _LPR) * 16, 16)] = jnp.ones(
            (16,), jnp.float32)
        return 0
    lax.fori_loop(0, _CHUNK * _LPR, _fill_ones, 0)

    def _fill_zero(i, _):
        zbuf_v[i // _LPR, pl.ds((i % _LPR) * 16, 16)] = jnp.zeros(
            (16,), jnp.float32)
        return 0
    lax.fori_loop(0, _ZR * _LPR, _fill_zero, 0)

    def _zero(j, _):
        pltpu.sync_copy(zbuf_v, acc_sh.at[pl.ds(s * _RPT + j * _ZR, _ZR)])
        return 0
    lax.fori_loop(0, _RPT // _ZR, _zero, 0)
    plsc.subcore_barrier()

    def _blk(b, _):
        pltpu.sync_copy(dst_hbm.at[c, s, pl.ds(b * _IB, _IB)], didx_v)

        def _scat(j, _):
            pltpu.sync_copy(ones_v, acc_sh.at[didx_v.at[j]], add=True)
            return 0
        lax.fori_loop(0, _IB, _scat, 0)
        return 0
    lax.fori_loop(0, _NBLK, _blk, 0)
    plsc.subcore_barrier()

    pltpu.sync_copy(acc_sh.at[pl.ds(s * _RPT, _RPT)],
                    out_hbm.at[c, pl.ds(s * _RPT, _RPT)])


_deg_call = pl.kernel(
    _deg_body,
    out_type=jax.ShapeDtypeStruct((_NC, _NPAD, _DW), jnp.float32),
    mesh=_MESH,
    scratch_types=[
        pltpu.VMEM((_IB, _CHUNK), jnp.int32),
        pltpu.VMEM((_CHUNK, _DW), jnp.float32),
        pltpu.VMEM((_ZR, _DW), jnp.float32),
        pltpu.VMEM_SHARED((_NPAD, _DW), jnp.float32),
        pltpu.SemaphoreType.DMA,
    ],
)


def _prop_body(h_hbm, src_hbm, dst_hbm, out_hbm,
               sidx_v, didx_v, rows_v, zbuf_v, acc_sh, sem, sem2):
    c = lax.axis_index("c")
    s = lax.axis_index("s")

    def _fill_zero(i, _):
        r = i // 8
        l = (i % 8) * 16
        zbuf_v[r, pl.ds(l, 16)] = jnp.zeros((16,), jnp.float32)
        return 0
    lax.fori_loop(0, _ZR * 8, _fill_zero, 0)

    def _zero(j, _):
        pltpu.sync_copy(zbuf_v, acc_sh.at[pl.ds(s * _RPT + j * _ZR, _ZR)])
        return 0
    lax.fori_loop(0, _RPT // _ZR, _zero, 0)
    plsc.subcore_barrier()

    def _blk(b, _):
        pltpu.sync_copy(src_hbm.at[c, s, pl.ds(b * _IB, _IB)], sidx_v)
        pltpu.sync_copy(dst_hbm.at[c, s, pl.ds(b * _IB, _IB)], didx_v)
        pltpu.async_copy(h_hbm.at[sidx_v.at[0]], rows_v.at[0], sem)

        def _edge(j, _):
            @pl.when(b * _IB + j >= 1)
            def _ws():
                pltpu.make_async_copy(rows_v.at[0],
                                      acc_sh.at[didx_v.at[0]], sem2).wait()

            @pl.when(j + 1 < _IB)
            def _pref():
                pltpu.async_copy(h_hbm.at[sidx_v.at[j + 1]],
                                 rows_v.at[(j + 1) % 2], sem)
            pltpu.make_async_copy(h_hbm.at[sidx_v.at[j]],
                                  rows_v.at[j % 2], sem).wait()
            pltpu.async_copy(rows_v.at[j % 2], acc_sh.at[didx_v.at[j]],
                             sem2, add=True)
            return 0
        lax.fori_loop(0, _IB, _edge, 0)
        return 0
    lax.fori_loop(0, _NBLK, _blk, 0)
    pltpu.make_async_copy(rows_v.at[0], acc_sh.at[didx_v.at[0]], sem2).wait()
    plsc.subcore_barrier()

    pltpu.sync_copy(acc_sh.at[pl.ds(s * _RPT, _RPT)],
                    out_hbm.at[c, pl.ds(s * _RPT, _RPT)])


_prop_call = pl.kernel(
    _prop_body,
    out_type=jax.ShapeDtypeStruct((_NC, _NPAD, _H), jnp.float32),
    mesh=_MESH,
    scratch_types=[
        pltpu.VMEM((_IB, _CHUNK), jnp.int32),
        pltpu.VMEM((_IB, _CHUNK), jnp.int32),
        pltpu.VMEM((2, _CHUNK, _H), jnp.float32),
        pltpu.VMEM((_ZR, _H), jnp.float32),
        pltpu.VMEM_SHARED((_NPAD, _H), jnp.float32),
        pltpu.SemaphoreType.DMA,
        pltpu.SemaphoreType.DMA,
    ],
)


def _tc1_body(x_ref, w1_ref, b1_ref, wka_ref, wkb_ref, wr_ref, br_ref,
              wwr_ref, deg_ref, h_ref, dinv_ref, part_ref):
    r0 = jnp.maximum(
        jnp.dot(x_ref[...], w1_ref[...], preferred_element_type=jnp.float32)
        + b1_ref[...], 0.0)
    deg = deg_ref[...]
    dinv = jnp.where(deg > 0, 1.0 / jnp.sqrt(deg), 0.0)
    dinv_ref[...] = dinv
    h_ref[0] = dinv[:, 0:1] * jnp.dot(
        r0, wka_ref[...], preferred_element_type=jnp.float32)
    h_ref[1] = dinv[:, 1:2] * jnp.dot(
        r0, wkb_ref[...], preferred_element_type=jnp.float32)
    part_ref[...] = wwr_ref[0, 0] * (
        jnp.dot(r0, wr_ref[...], preferred_element_type=jnp.float32)
        + br_ref[0, 0])


_tc1_call = pl.pallas_call(
    _tc1_body,
    grid=(_GRID,),
    in_specs=[
        pl.BlockSpec((_BR, _H), lambda i: (i, 0)),
        pl.BlockSpec((_H, _H), lambda i: (0, 0)),
        pl.BlockSpec((1, _H), lambda i: (0, 0)),
        pl.BlockSpec((_H, _H), lambda i: (0, 0)),
        pl.BlockSpec((_H, _H), lambda i: (0, 0)),
        pl.BlockSpec((_H, 1), lambda i: (0, 0)),
        pl.BlockSpec((1, 1), lambda i: (0, 0)),
        pl.BlockSpec((1, 1), lambda i: (0, 0)),
        pl.BlockSpec((_BR, _NC), lambda i: (i, 0)),
    ],
    out_specs=[
        pl.BlockSpec((_NC, _BR, _H), lambda i: (0, i, 0)),
        pl.BlockSpec((_BR, _NC), lambda i: (i, 0)),
        pl.BlockSpec((_BR, 1), lambda i: (i, 0)),
    ],
    out_shape=[
        jax.ShapeDtypeStruct((_NC, _N, _H), jnp.float32),
        jax.ShapeDtypeStruct((_N, _NC), jnp.float32),
        jax.ShapeDtypeStruct((_N, 1), jnp.float32),
    ],
)


def _mid_body(s_ref, dinv_ref, ba_ref, bb_ref, wka_ref, wkb_ref, wr_ref,
              br_ref, wwr_ref, pin_ref, h_ref, part_ref):
    dinv = dinv_ref[...]
    left = dinv[:, 0:1] * s_ref[0] + ba_ref[...]
    right = dinv[:, 1:2] * s_ref[1] + bb_ref[...]
    rk = jnp.concatenate([left, right], axis=1)
    h_ref[0] = dinv[:, 0:1] * jnp.dot(
        rk, wka_ref[...], preferred_element_type=jnp.float32)
    h_ref[1] = dinv[:, 1:2] * jnp.dot(
        rk, wkb_ref[...], preferred_element_type=jnp.float32)
    part_ref[...] = pin_ref[...] + wwr_ref[0, 0] * (
        jnp.dot(rk, wr_ref[...], preferred_element_type=jnp.float32)
        + br_ref[0, 0])


_mid_call = pl.pallas_call(
    _mid_body,
    grid=(_GRID,),
    in_specs=[
        pl.BlockSpec((_NC, _BR, _H), lambda i: (0, i, 0)),
        pl.BlockSpec((_BR, _NC), lambda i: (i, 0)),
        pl.BlockSpec((1, _H), lambda i: (0, 0)),
        pl.BlockSpec((1, _H), lambda i: (0, 0)),
        pl.BlockSpec((2 * _H, _H), lambda i: (0, 0)),
        pl.BlockSpec((2 * _H, _H), lambda i: (0, 0)),
        pl.BlockSpec((2 * _H, 1), lambda i: (0, 0)),
        pl.BlockSpec((1, 1), lambda i: (0, 0)),
        pl.BlockSpec((1, 1), lambda i: (0, 0)),
        pl.BlockSpec((_BR, 1), lambda i: (i, 0)),
    ],
    out_specs=[
        pl.BlockSpec((_NC, _BR, _H), lambda i: (0, i, 0)),
        pl.BlockSpec((_BR, 1), lambda i: (i, 0)),
    ],
    out_shape=[
        jax.ShapeDtypeStruct((_NC, _N, _H), jnp.float32),
        jax.ShapeDtypeStruct((_N, 1), jnp.float32),
    ],
)


def _tc4_body(s_ref, dinv_ref, ba_ref, bb_ref, wr_ref, br_ref, wwr_ref,
              pin_ref, out_ref):
    dinv = dinv_ref[...]
    left = dinv[:, 0:1] * s_ref[0] + ba_ref[...]
    right = dinv[:, 1:2] * s_ref[1] + bb_ref[...]
    rk = jnp.concatenate([left, right], axis=1)
    out_ref[...] = pin_ref[...] + wwr_ref[0, 0] * (
        jnp.dot(rk, wr_ref[...], preferred_element_type=jnp.float32)
        + br_ref[0, 0])


_tc4_call = pl.pallas_call(
    _tc4_body,
    grid=(_GRID,),
    in_specs=[
        pl.BlockSpec((_NC, _BR, _H), lambda i: (0, i, 0)),
        pl.BlockSpec((_BR, _NC), lambda i: (i, 0)),
        pl.BlockSpec((1, _H), lambda i: (0, 0)),
        pl.BlockSpec((1, _H), lambda i: (0, 0)),
        pl.BlockSpec((2 * _H, 1), lambda i: (0, 0)),
        pl.BlockSpec((1, 1), lambda i: (0, 0)),
        pl.BlockSpec((1, 1), lambda i: (0, 0)),
        pl.BlockSpec((_BR, 1), lambda i: (i, 0)),
    ],
    out_specs=pl.BlockSpec((_BR, 1), lambda i: (i, 0)),
    out_shape=jax.ShapeDtypeStruct((_N, 1), jnp.float32),
)


def kernel(x, edge_index, edge_index_aux, W1, b1, Wk11, bk11, Wk12, bk12,
           Wk21, bk21, Wk22, bk22, Wk31, bk31, Wk32, bk32, Wr0, br0, Wr1,
           br1, Wr2, br2, Wr3, br3, wr0, wr1, wr2, wr3):
    s1, d1 = edge_index[0], edge_index[1]
    s2, d2 = edge_index_aux[0], edge_index_aux[1]
    dst4 = jnp.stack([d1, d2]).reshape(_NC, _NS, _NCHUNK, _CHUNK)
    src4 = jnp.stack([s1, s2 + _N]).reshape(_NC, _NS, _NCHUNK, _CHUNK)

    deg = _deg_call(dst4)[:, :_N, 0].T

    h, dinv, part = _tc1_call(
        x, W1, b1.reshape(1, _H), Wk11, Wk12, Wr0,
        br0.reshape(1, 1), wr0.reshape(1, 1), deg)
    S = _prop_call(h.reshape(_NC * _N, _H), src4, dst4)

    h, part = _mid_call(
        S, dinv, bk11.reshape(1, _H), bk12.reshape(1, _H), Wk21, Wk22, Wr1,
        br1.reshape(1, 1), wr1.reshape(1, 1), part)
    S = _prop_call(h.reshape(_NC * _N, _H), src4, dst4)

    h, part = _mid_call(
        S, dinv, bk21.reshape(1, _H), bk22.reshape(1, _H), Wk31, Wk32, Wr2,
        br2.reshape(1, 1), wr2.reshape(1, 1), part)
    S = _prop_call(h.reshape(_NC * _N, _H), src4, dst4)

    out = _tc4_call(
        S, dinv, bk31.reshape(1, _H), bk32.reshape(1, _H), Wr3,
        br3.reshape(1, 1), wr3.reshape(1, 1), part)
    return out


# speedup vs baseline: 19.3893x; 1.0852x over previous
import jax
import jax.numpy as jnp
from jax import lax
from jax.experimental import pallas as pl
from jax.experimental.pallas import tpu as pltpu
from jax.experimental.pallas import tpu_sc as plsc

_N = 10000
_E = 320000
_H = 128
_NC = 2
_NS = 16
_CHUNK = 125
_NCHUNK = (_E // _NS) // _CHUNK
_NPAD = 10240
_RPT = _NPAD // _NS
_ZR = 16
_IB = 40
_NBLK = _NCHUNK // _IB
_BR = 1000
_GRID = _N // _BR

_MESH = plsc.VectorSubcoreMesh(core_axis_name="c", subcore_axis_name="s")


_DW = 128


def _deg_body(dst_hbm, out_hbm, didx_v, ones_v, zbuf_v, acc_sh, sem):
    c = lax.axis_index("c")
    s = lax.axis_index("s")

    _LPR = _DW // 16

    def _fill_ones(i, _):
        ones_v[i // _LPR, pl.ds((i % _LPR) * 16, 16)] = jnp.ones(
            (16,), jnp.float32)
        return 0
    lax.fori_loop(0, _CHUNK * _LPR, _fill_ones, 0)

    def _fill_zero(i, _):
        zbuf_v[i // _LPR, pl.ds((i % _LPR) * 16, 16)] = jnp.zeros(
            (16,), jnp.float32)
        return 0
    lax.fori_loop(0, _ZR * _LPR, _fill_zero, 0)

    def _zero(j, _):
        pltpu.sync_copy(zbuf_v, acc_sh.at[pl.ds(s * _RPT + j * _ZR, _ZR)])
        return 0
    lax.fori_loop(0, _RPT // _ZR, _zero, 0)
    plsc.subcore_barrier()

    def _blk(b, _):
        @pl.when(b >= 2)
        def _dr():
            def _w(j, _):
                pltpu.make_async_copy(ones_v, acc_sh.at[didx_v.at[0, 0]],
                                      sem).wait()
                return 0
            lax.fori_loop(0, _IB, _w, 0)
        pltpu.sync_copy(dst_hbm.at[c, s, pl.ds(b * _IB, _IB)],
                        didx_v.at[b % 2])

        def _scat(j, _):
            pltpu.async_copy(ones_v, acc_sh.at[didx_v.at[b % 2, j]],
                             sem, add=True)
            return 0
        lax.fori_loop(0, _IB, _scat, 0)
        return 0
    lax.fori_loop(0, _NBLK, _blk, 0)

    def _wend(j, _):
        pltpu.make_async_copy(ones_v, acc_sh.at[didx_v.at[0, 0]], sem).wait()
        return 0
    lax.fori_loop(0, 2 * _IB, _wend, 0)
    plsc.subcore_barrier()

    pltpu.sync_copy(acc_sh.at[pl.ds(s * _RPT, _RPT)],
                    out_hbm.at[c, pl.ds(s * _RPT, _RPT)])


_deg_call = pl.kernel(
    _deg_body,
    out_type=jax.ShapeDtypeStruct((_NC, _NPAD, _DW), jnp.float32),
    mesh=_MESH,
    scratch_types=[
        pltpu.VMEM((2, _IB, _CHUNK), jnp.int32),
        pltpu.VMEM((_CHUNK, _DW), jnp.float32),
        pltpu.VMEM((_ZR, _DW), jnp.float32),
        pltpu.VMEM_SHARED((_NPAD, _DW), jnp.float32),
        pltpu.SemaphoreType.DMA,
    ],
)


def _prop_body(h_hbm, src_hbm, dst_hbm, out_hbm,
               sidx_v, didx_v, rows_v, zbuf_v, acc_sh, sem, sem2):
    c = lax.axis_index("c")
    s = lax.axis_index("s")

    def _fill_zero(i, _):
        r = i // 8
        l = (i % 8) * 16
        zbuf_v[r, pl.ds(l, 16)] = jnp.zeros((16,), jnp.float32)
        return 0
    lax.fori_loop(0, _ZR * 8, _fill_zero, 0)

    def _zero(j, _):
        pltpu.sync_copy(zbuf_v, acc_sh.at[pl.ds(s * _RPT + j * _ZR, _ZR)])
        return 0
    lax.fori_loop(0, _RPT // _ZR, _zero, 0)
    plsc.subcore_barrier()

    def _blk(b, _):
        pltpu.sync_copy(src_hbm.at[c, s, pl.ds(b * _IB, _IB)], sidx_v)
        pltpu.sync_copy(dst_hbm.at[c, s, pl.ds(b * _IB, _IB)], didx_v)
        pltpu.async_copy(h_hbm.at[sidx_v.at[0]], rows_v.at[0], sem)

        def _edge(j, _):
            @pl.when(b * _IB + j >= 1)
            def _ws():
                pltpu.make_async_copy(rows_v.at[0],
                                      acc_sh.at[didx_v.at[0]], sem2).wait()

            @pl.when(j + 1 < _IB)
            def _pref():
                pltpu.async_copy(h_hbm.at[sidx_v.at[j + 1]],
                                 rows_v.at[(j + 1) % 2], sem)
            pltpu.make_async_copy(h_hbm.at[sidx_v.at[j]],
                                  rows_v.at[j % 2], sem).wait()
            pltpu.async_copy(rows_v.at[j % 2], acc_sh.at[didx_v.at[j]],
                             sem2, add=True)
            return 0
        lax.fori_loop(0, _IB, _edge, 0)
        return 0
    lax.fori_loop(0, _NBLK, _blk, 0)
    pltpu.make_async_copy(rows_v.at[0], acc_sh.at[didx_v.at[0]], sem2).wait()
    plsc.subcore_barrier()

    pltpu.sync_copy(acc_sh.at[pl.ds(s * _RPT, _RPT)],
                    out_hbm.at[c, pl.ds(s * _RPT, _RPT)])


_prop_call = pl.kernel(
    _prop_body,
    out_type=jax.ShapeDtypeStruct((_NC, _NPAD, _H), jnp.float32),
    mesh=_MESH,
    scratch_types=[
        pltpu.VMEM((_IB, _CHUNK), jnp.int32),
        pltpu.VMEM((_IB, _CHUNK), jnp.int32),
        pltpu.VMEM((2, _CHUNK, _H), jnp.float32),
        pltpu.VMEM((_ZR, _H), jnp.float32),
        pltpu.VMEM_SHARED((_NPAD, _H), jnp.float32),
        pltpu.SemaphoreType.DMA,
        pltpu.SemaphoreType.DMA,
    ],
)


def _tc1_body(x_ref, w1_ref, b1_ref, wka_ref, wkb_ref, wr_ref, br_ref,
              wwr_ref, deg_ref, h_ref, dinv_ref, part_ref):
    r0 = jnp.maximum(
        jnp.dot(x_ref[...], w1_ref[...], preferred_element_type=jnp.float32)
        + b1_ref[...], 0.0)
    deg = jnp.concatenate(
        [deg_ref[0, :, 0:1], deg_ref[1, :, 0:1]], axis=1)
    dinv = jnp.where(deg > 0, 1.0 / jnp.sqrt(deg), 0.0)
    dinv_ref[...] = dinv
    h_ref[0] = dinv[:, 0:1] * jnp.dot(
        r0, wka_ref[...], preferred_element_type=jnp.float32)
    h_ref[1] = dinv[:, 1:2] * jnp.dot(
        r0, wkb_ref[...], preferred_element_type=jnp.float32)
    part_ref[...] = wwr_ref[0, 0] * (
        jnp.dot(r0, wr_ref[...], preferred_element_type=jnp.float32)
        + br_ref[0, 0])


_tc1_call = pl.pallas_call(
    _tc1_body,
    grid=(_GRID,),
    in_specs=[
        pl.BlockSpec((_BR, _H), lambda i: (i, 0)),
        pl.BlockSpec((_H, _H), lambda i: (0, 0)),
        pl.BlockSpec((1, _H), lambda i: (0, 0)),
        pl.BlockSpec((_H, _H), lambda i: (0, 0)),
        pl.BlockSpec((_H, _H), lambda i: (0, 0)),
        pl.BlockSpec((_H, 1), lambda i: (0, 0)),
        pl.BlockSpec((1, 1), lambda i: (0, 0)),
        pl.BlockSpec((1, 1), lambda i: (0, 0)),
        pl.BlockSpec((_NC, _BR, _DW), lambda i: (0, i, 0)),
    ],
    out_specs=[
        pl.BlockSpec((_NC, _BR, _H), lambda i: (0, i, 0)),
        pl.BlockSpec((_BR, _NC), lambda i: (i, 0)),
        pl.BlockSpec((_BR, 1), lambda i: (i, 0)),
    ],
    out_shape=[
        jax.ShapeDtypeStruct((_NC, _N, _H), jnp.float32),
        jax.ShapeDtypeStruct((_N, _NC), jnp.float32),
        jax.ShapeDtypeStruct((_N, 1), jnp.float32),
    ],
)


def _mid_body(s_ref, dinv_ref, ba_ref, bb_ref, wka_ref, wkb_ref, wr_ref,
              br_ref, wwr_ref, pin_ref, h_ref, part_ref):
    dinv = dinv_ref[...]
    left = dinv[:, 0:1] * s_ref[0] + ba_ref[...]
    right = dinv[:, 1:2] * s_ref[1] + bb_ref[...]
    rk = jnp.concatenate([left, right], axis=1)
    h_ref[0] = dinv[:, 0:1] * jnp.dot(
        rk, wka_ref[...], preferred_element_type=jnp.float32)
    h_ref[1] = dinv[:, 1:2] * jnp.dot(
        rk, wkb_ref[...], preferred_element_type=jnp.float32)
    part_ref[...] = pin_ref[...] + wwr_ref[0, 0] * (
        jnp.dot(rk, wr_ref[...], preferred_element_type=jnp.float32)
        + br_ref[0, 0])


_mid_call = pl.pallas_call(
    _mid_body,
    grid=(_GRID,),
    in_specs=[
        pl.BlockSpec((_NC, _BR, _H), lambda i: (0, i, 0)),
        pl.BlockSpec((_BR, _NC), lambda i: (i, 0)),
        pl.BlockSpec((1, _H), lambda i: (0, 0)),
        pl.BlockSpec((1, _H), lambda i: (0, 0)),
        pl.BlockSpec((2 * _H, _H), lambda i: (0, 0)),
        pl.BlockSpec((2 * _H, _H), lambda i: (0, 0)),
        pl.BlockSpec((2 * _H, 1), lambda i: (0, 0)),
        pl.BlockSpec((1, 1), lambda i: (0, 0)),
        pl.BlockSpec((1, 1), lambda i: (0, 0)),
        pl.BlockSpec((_BR, 1), lambda i: (i, 0)),
    ],
    out_specs=[
        pl.BlockSpec((_NC, _BR, _H), lambda i: (0, i, 0)),
        pl.BlockSpec((_BR, 1), lambda i: (i, 0)),
    ],
    out_shape=[
        jax.ShapeDtypeStruct((_NC, _N, _H), jnp.float32),
        jax.ShapeDtypeStruct((_N, 1), jnp.float32),
    ],
)


def _tc4_body(s_ref, dinv_ref, ba_ref, bb_ref, wr_ref, br_ref, wwr_ref,
              pin_ref, out_ref):
    dinv = dinv_ref[...]
    left = dinv[:, 0:1] * s_ref[0] + ba_ref[...]
    right = dinv[:, 1:2] * s_ref[1] + bb_ref[...]
    rk = jnp.concatenate([left, right], axis=1)
    out_ref[...] = pin_ref[...] + wwr_ref[0, 0] * (
        jnp.dot(rk, wr_ref[...], preferred_element_type=jnp.float32)
        + br_ref[0, 0])


_tc4_call = pl.pallas_call(
    _tc4_body,
    grid=(_GRID,),
    in_specs=[
        pl.BlockSpec((_NC, _BR, _H), lambda i: (0, i, 0)),
        pl.BlockSpec((_BR, _NC), lambda i: (i, 0)),
        pl.BlockSpec((1, _H), lambda i: (0, 0)),
        pl.BlockSpec((1, _H), lambda i: (0, 0)),
        pl.BlockSpec((2 * _H, 1), lambda i: (0, 0)),
        pl.BlockSpec((1, 1), lambda i: (0, 0)),
        pl.BlockSpec((1, 1), lambda i: (0, 0)),
        pl.BlockSpec((_BR, 1), lambda i: (i, 0)),
    ],
    out_specs=pl.BlockSpec((_BR, 1), lambda i: (i, 0)),
    out_shape=jax.ShapeDtypeStruct((_N, 1), jnp.float32),
)


def kernel(x, edge_index, edge_index_aux, W1, b1, Wk11, bk11, Wk12, bk12,
           Wk21, bk21, Wk22, bk22, Wk31, bk31, Wk32, bk32, Wr0, br0, Wr1,
           br1, Wr2, br2, Wr3, br3, wr0, wr1, wr2, wr3):
    s1, d1 = edge_index[0], edge_index[1]
    s2, d2 = edge_index_aux[0], edge_index_aux[1]
    dst4 = jnp.stack([d1, d2]).reshape(_NC, _NS, _NCHUNK, _CHUNK)
    src4 = jnp.stack([s1, s2 + _N]).reshape(_NC, _NS, _NCHUNK, _CHUNK)

    deg = _deg_call(dst4)

    h, dinv, part = _tc1_call(
        x, W1, b1.reshape(1, _H), Wk11, Wk12, Wr0,
        br0.reshape(1, 1), wr0.reshape(1, 1), deg)
    S = _prop_call(h.reshape(_NC * _N, _H), src4, dst4)

    h, part = _mid_call(
        S, dinv, bk11.reshape(1, _H), bk12.reshape(1, _H), Wk21, Wk22, Wr1,
        br1.reshape(1, 1), wr1.reshape(1, 1), part)
    S = _prop_call(h.reshape(_NC * _N, _H), src4, dst4)

    h, part = _mid_call(
        S, dinv, bk21.reshape(1, _H), bk22.reshape(1, _H), Wk31, Wk32, Wr2,
        br2.reshape(1, 1), wr2.reshape(1, 1), part)
    S = _prop_call(h.reshape(_NC * _N, _H), src4, dst4)

    out = _tc4_call(
        S, dinv, bk31.reshape(1, _H), bk32.reshape(1, _H), Wr3,
        br3.reshape(1, 1), wr3.reshape(1, 1), part)
    return out
